# baseline (speedup 1.0000x reference)
import jax
import jax.numpy as jnp
from jax.experimental import pallas as pl
from jax.experimental.pallas import tpu as pltpu

_P = 32
_B = 64
_N = 784
_NE = _N + 2
_STEPS = 7
_G = _P // 2


def _roll_up(v):
    return jnp.concatenate([v[-1:], v[:-1]], axis=0)


def _roll_dn(v):
    return jnp.concatenate([v[1:], v[:1]], axis=0)


def _dadmm_body(hs_ref, x_ref, y_ref, a_ref, w_ref, a_out, w_out,
                A_ref, MU_ref, X_ref):
    X_ref[:, :, :_N] = x_ref[...]
    X_ref[:, :, _N:_NE] = jnp.ones_like(X_ref[:, :, _N:_NE])
    A_ref[:, :, :_N] = a_ref[...]
    A_ref[:, :, _N:_N + 1] = w_ref[...]
    A_ref[:, :, _N + 1:_NE] = -y_ref[...]
    MU_ref[...] = jnp.zeros_like(MU_ref)

    lane = jax.lax.broadcasted_iota(jnp.int32, (1, 1, _NE), 2)
    is_a = lane < _N
    is_w = lane == _N

    def hvec(ha, hw):
        return jnp.where(is_a, ha, jnp.where(is_w, hw, 0.0))

    EV = slice(0, _B)
    OD = slice(_B, 2 * _B)

    def phase(act, nbr, roll, k, dual_prev):
        x = X_ref[:, act, :]
        A = A_ref[:, act, :]
        An = A_ref[:, nbr, :]
        nsum = roll(An) + An
        if dual_prev and k > 0:
            H3p = hvec(hs_ref[k - 1, 3], hs_ref[k - 1, 4])
            MU_ref[:, act, :] += H3p * (2.0 * A - nsum)
        H0 = hvec(hs_ref[k, 0], hs_ref[k, 2])
        H1 = hvec(hs_ref[k, 1], hs_ref[k, 5])
        ones = jnp.ones((_NE, 1), jnp.float32)
        c = jax.lax.dot_general(
            x * A, ones, (((2,), (0,)), ((), ())),
            preferred_element_type=jnp.float32)
        A_new = A - H1 * (c * x + 2.0 * H0 * A
                          + 2.0 * MU_ref[:, act, :] - H0 * nsum)
        A_ref[:, act, :] = A_new
        if not dual_prev:
            H3 = hvec(hs_ref[k, 3], hs_ref[k, 4])
            MU_ref[:, act, :] += H3 * (2.0 * A_new - nsum)

    for k in range(_STEPS):
        phase(EV, OD, _roll_up, k, dual_prev=True)
        phase(OD, EV, _roll_dn, k, dual_prev=False)

    a_out[...] = A_ref[:, :, :_N]
    w_out[...] = A_ref[:, :, _N:_N + 1]


def kernel(inputs, labels, a0, omega0, hyp, no_hyp, neighbors):
    del neighbors
    x = inputs.reshape(_G, 2 * _B, _N)
    y = labels.reshape(_G, 2 * _B, 1)
    a_in = a0.reshape(_G, 2 * _B, _N)
    w_in = omega0.reshape(_G, 2 * _B, 1)
    hs = jnp.abs(jnp.concatenate([no_hyp, hyp], axis=0))

    a_fin, w_fin = pl.pallas_call(
        _dadmm_body,
        out_shape=[
            jax.ShapeDtypeStruct((_G, 2 * _B, _N), jnp.float32),
            jax.ShapeDtypeStruct((_G, 2 * _B, 1), jnp.float32),
        ],
        in_specs=[
            pl.BlockSpec(memory_space=pltpu.SMEM),
            pl.BlockSpec(memory_space=pltpu.VMEM),
            pl.BlockSpec(memory_space=pltpu.VMEM),
            pl.BlockSpec(memory_space=pltpu.VMEM),
            pl.BlockSpec(memory_space=pltpu.VMEM),
        ],
        out_specs=[
            pl.BlockSpec(memory_space=pltpu.VMEM),
            pl.BlockSpec(memory_space=pltpu.VMEM),
        ],
        scratch_shapes=[
            pltpu.VMEM((_G, 2 * _B, _NE), jnp.float32),
            pltpu.VMEM((_G, 2 * _B, _NE), jnp.float32),
            pltpu.VMEM((_G, 2 * _B, _NE), jnp.float32),
        ],
    )(hs, x, y, a_in, w_in)

    return (a_fin.reshape(_P, _B, _N, 1), w_fin.reshape(_P, _B, 1, 1))


# speedup vs baseline: 1.0152x; 1.0152x over previous
import jax
import jax.numpy as jnp
from jax.experimental import pallas as pl
from jax.experimental.pallas import tpu as pltpu

_P = 32
_B = 64
_N = 784
_NE = _N + 2
_STEPS = 7
_G = _P // 2


def _roll_up(v):
    return jnp.concatenate([v[-1:], v[:-1]], axis=0)


def _roll_dn(v):
    return jnp.concatenate([v[1:], v[:1]], axis=0)


def _dadmm_body(hs_ref, x_ref, y_ref, a_ref, w_ref, a_out, w_out,
                A_ref, MU_ref, X_ref):
    X_ref[:, :, :_N] = x_ref[...]
    X_ref[:, :, _N:_NE] = jnp.ones_like(X_ref[:, :, _N:_NE])
    A_ref[:, :, :_N] = a_ref[...]
    A_ref[:, :, _N:_N + 1] = w_ref[...]
    A_ref[:, :, _N + 1:_NE] = -y_ref[...]

    lane = jax.lax.broadcasted_iota(jnp.int32, (1, 1, _NE), 2)
    is_a = lane < _N
    is_w = lane == _N

    def hvec(ha, hw):
        return jnp.where(is_a, ha, jnp.where(is_w, hw, 0.0))

    EV = slice(0, _B)
    OD = slice(_B, 2 * _B)

    def phase(act, nbr, roll, k, dual_prev):
        x = X_ref[:, act, :]
        A = A_ref[:, act, :]
        An = A_ref[:, nbr, :]
        nsum = roll(An) + An
        if dual_prev and k > 0:
            H3p = hvec(hs_ref[k - 1, 3], hs_ref[k - 1, 4])
            dual = H3p * (2.0 * A - nsum)
            if k == 1:
                MU_ref[:, act, :] = dual
            else:
                MU_ref[:, act, :] += dual
        H0 = hvec(hs_ref[k, 0], hs_ref[k, 2])
        H1 = hvec(hs_ref[k, 1], hs_ref[k, 5])
        c = jnp.sum(x * A, axis=-1, keepdims=True)
        g = c * x + 2.0 * H0 * A - H0 * nsum
        if k > 0:
            g = g + 2.0 * MU_ref[:, act, :]
        A_new = A - H1 * g
        A_ref[:, act, :] = A_new
        if not dual_prev:
            H3 = hvec(hs_ref[k, 3], hs_ref[k, 4])
            dual = H3 * (2.0 * A_new - nsum)
            if k == 0:
                MU_ref[:, act, :] = dual
            else:
                MU_ref[:, act, :] += dual

    for k in range(_STEPS):
        phase(EV, OD, _roll_up, k, dual_prev=True)
        phase(OD, EV, _roll_dn, k, dual_prev=False)

    a_out[...] = A_ref[:, :, :_N]
    w_out[...] = A_ref[:, :, _N:_N + 1]


def kernel(inputs, labels, a0, omega0, hyp, no_hyp, neighbors):
    del neighbors
    x = inputs.reshape(_G, 2 * _B, _N)
    y = labels.reshape(_G, 2 * _B, 1)
    a_in = a0.reshape(_G, 2 * _B, _N)
    w_in = omega0.reshape(_G, 2 * _B, 1)
    hs = jnp.abs(jnp.concatenate([no_hyp, hyp], axis=0))

    a_fin, w_fin = pl.pallas_call(
        _dadmm_body,
        out_shape=[
            jax.ShapeDtypeStruct((_G, 2 * _B, _N), jnp.float32),
            jax.ShapeDtypeStruct((_G, 2 * _B, 1), jnp.float32),
        ],
        in_specs=[
            pl.BlockSpec(memory_space=pltpu.SMEM),
            pl.BlockSpec(memory_space=pltpu.VMEM),
            pl.BlockSpec(memory_space=pltpu.VMEM),
            pl.BlockSpec(memory_space=pltpu.VMEM),
            pl.BlockSpec(memory_space=pltpu.VMEM),
        ],
        out_specs=[
            pl.BlockSpec(memory_space=pltpu.VMEM),
            pl.BlockSpec(memory_space=pltpu.VMEM),
        ],
        scratch_shapes=[
            pltpu.VMEM((_G, 2 * _B, _NE), jnp.float32),
            pltpu.VMEM((_G, 2 * _B, _NE), jnp.float32),
            pltpu.VMEM((_G, 2 * _B, _NE), jnp.float32),
        ],
    )(hs, x, y, a_in, w_in)

    return (a_fin.reshape(_P, _B, _N, 1), w_fin.reshape(_P, _B, 1, 1))
